# baseline (speedup 1.0000x reference)
import functools

import jax
import jax.numpy as jnp
from jax import lax
from jax.experimental import pallas as pl
from jax.experimental.pallas import tpu as pltpu
from jax.experimental.pallas import tpu_sc as plsc

NUM_CORES = 2
NUM_SUBCORES = 16
NUM_WORKERS = NUM_CORES * NUM_SUBCORES


def kernel(tokens, table, register_tokens):
    B, S = tokens.shape
    V, D = table.shape
    R = register_tokens.shape[0]
    T = R + S
    rows_per_w = B // NUM_WORKERS
    half = S // 2

    tokens3 = tokens.reshape(B, 2, half).astype(jnp.int32)
    mesh = plsc.VectorSubcoreMesh(core_axis_name="c", subcore_axis_name="s")

    @functools.partial(
        pl.kernel,
        mesh=mesh,
        out_type=jax.ShapeDtypeStruct((B, T, D), jnp.float32),
        compiler_params=pltpu.CompilerParams(use_tc_tiling_on_sc=False),
        scratch_types=[
            pltpu.VMEM((2, half), jnp.int32),
            pltpu.VMEM((T, D), jnp.float32),
            pltpu.SemaphoreType.DMA,
        ],
    )
    def emb(tokens_hbm, table_hbm, regs_hbm, out_hbm, idx_v, rows_v, sem):
        wid = lax.axis_index("s") * NUM_CORES + lax.axis_index("c")
        base = wid * rows_per_w
        pltpu.sync_copy(regs_hbm, rows_v.at[pl.ds(0, R)])

        def body(i, _):
            b = base + i
            pltpu.sync_copy(tokens_hbm.at[b], idx_v)
            cp1 = pltpu.async_copy(
                table_hbm.at[idx_v.at[0]], rows_v.at[pl.ds(R, half)], sem
            )
            cp2 = pltpu.async_copy(
                table_hbm.at[idx_v.at[1]], rows_v.at[pl.ds(R + half, half)], sem
            )
            cp1.wait()
            cp2.wait()
            pltpu.sync_copy(rows_v, out_hbm.at[b])
            return ()

        lax.fori_loop(0, rows_per_w, body, ())

    return emb(tokens3, table, register_tokens)


# speedup vs baseline: 1.1206x; 1.1206x over previous
import functools

import jax
import jax.numpy as jnp
from jax import lax
from jax.experimental import pallas as pl
from jax.experimental.pallas import tpu as pltpu
from jax.experimental.pallas import tpu_sc as plsc

NUM_CORES = 2
NUM_SUBCORES = 16
NUM_WORKERS = NUM_CORES * NUM_SUBCORES
NBUF = 4
LAG = 2


def kernel(tokens, table, register_tokens):
    B, S = tokens.shape
    V, D = table.shape
    R = register_tokens.shape[0]
    T = R + S
    RPW = B // NUM_WORKERS
    NG = RPW // NBUF
    half = S // 2

    tokens3 = tokens.reshape(B, 2, half).astype(jnp.int32)
    mesh = plsc.VectorSubcoreMesh(core_axis_name="c", subcore_axis_name="s")

    @functools.partial(
        pl.kernel,
        mesh=mesh,
        out_type=jax.ShapeDtypeStruct((B, T, D), jnp.float32),
        compiler_params=pltpu.CompilerParams(use_tc_tiling_on_sc=False),
        scratch_types=[
            pltpu.VMEM((RPW, 2, half), jnp.int32),
            pltpu.VMEM((NBUF, T, D), jnp.float32),
            pltpu.SemaphoreType.DMA((NBUF,)),
            pltpu.SemaphoreType.DMA((NBUF,)),
        ],
    )
    def emb(tokens_hbm, table_hbm, regs_hbm, out_hbm, idx_all, rows_v, gsem, osem):
        wid = lax.axis_index("s") * NUM_CORES + lax.axis_index("c")
        base = wid * RPW
        pltpu.sync_copy(tokens_hbm.at[pl.ds(base, RPW)], idx_all)
        for b in range(NBUF):
            pltpu.sync_copy(regs_hbm, rows_v.at[b, pl.ds(0, R)])

        def gather_copies(slot, i):
            return (
                pltpu.make_async_copy(
                    table_hbm.at[idx_all.at[i, 0]],
                    rows_v.at[slot, pl.ds(R, half)],
                    gsem.at[slot],
                ),
                pltpu.make_async_copy(
                    table_hbm.at[idx_all.at[i, 1]],
                    rows_v.at[slot, pl.ds(R + half, half)],
                    gsem.at[slot],
                ),
            )

        def out_copy(slot, i):
            return pltpu.make_async_copy(
                rows_v.at[slot], out_hbm.at[base + i], osem.at[slot]
            )

        def fire_gather(slot, i):
            for cp in gather_copies(slot, i):
                cp.start()

        def wait_gather(slot, i):
            for cp in gather_copies(slot, i):
                cp.wait()

        for b in range(NBUF):
            fire_gather(b, b)

        def body(g, _):
            i0 = g * NBUF

            def refill(slot):
                out_copy(slot, i0 + slot).wait()

                @pl.when(g + 1 < NG)
                def _():
                    fire_gather(slot, i0 + NBUF + slot)

            for b in range(NBUF):
                wait_gather(b, i0 + b)
                out_copy(b, i0 + b).start()
                if b >= LAG:
                    refill(b - LAG)
            for b in range(NBUF - LAG, NBUF):
                refill(b)
            return ()

        lax.fori_loop(0, NG, body, ())

    return emb(tokens3, table, register_tokens)
